# baseline (speedup 1.0000x reference)
import jax
import jax.numpy as jnp
from jax import lax
from jax.experimental import pallas as pl
from jax.experimental.pallas import tpu as pltpu
from jax.experimental.pallas import tpu_sc as plsc

E = 800_000
D = 64
CH = 1024
BLK = 128
NCHUNKS = 800
EPAD = NCHUNKS * CH
CHUNKS_PER_SUB = NCHUNKS // 16
QUARTER = 12_500
NPASS = 2
ACC_ROWS = 12_544
ROWS_PER_SUB = ACC_ROWS // 16
CNT_ROWS = 896
IDXROWS = CH // 128
NSB = 3
ALPHA = 0.6
BETA = 0.3


def _sc_body(ent_t, rel_t, pidx_h, zmsg_h, zcnt_h,
             msg_out, cnt_out,
             pidx_v, eidx_c, ridx_c, midx_c,
             ent_w, rel_w, hist, iota_v, acc, cacc, sem_g, sem_s):
    c = lax.axis_index("c")
    s = lax.axis_index("s")
    chunk0 = s * CHUNKS_PER_SUB
    i32 = jnp.int32
    lane = lax.iota(i32, 16)
    ones16 = jnp.full((16,), 1.0, jnp.float32)
    zero16 = jnp.zeros((16,), i32)
    dummy16 = jnp.full((16,), QUARTER, i32)

    for k in range(CNT_ROWS // 16):
        iota_v[pl.ds(k * 16, 16)] = lane + (k * 16)

    for p in range(NPASS):
        q = c * NPASS + p
        base = q * QUARTER

        pltpu.sync_copy(zmsg_h, acc.at[pl.ds(s * ROWS_PER_SUB, ROWS_PER_SUB)])
        pltpu.sync_copy(zcnt_h, hist)

        @pl.when(s == 0)
        def _():
            pltpu.sync_copy(zcnt_h, cacc)

        plsc.subcore_barrier()

        @pl.loop(0, CHUNKS_PER_SUB)
        def chunkloop(g):
            chunk = chunk0 + g
            pltpu.sync_copy(pidx_h.at[pl.ds(2 * IDXROWS * chunk, 2 * IDXROWS)],
                            pidx_v)

            @pl.loop(0, CH // 16, init_carry=0)
            def compress(k, off):
                r = k // 8
                sl = pl.ds((k % 8) * 16, 16)
                w0 = pidx_v[r, sl]
                loc = lax.bitwise_and(w0, 0x1FFFF) - base
                valid = (loc >= 0) & (loc < QUARTER)
                cs = plsc.cumsum(jnp.where(valid, 1, 0))
                pos = jnp.where(valid, off + cs - 1, CH + NSB * BLK + lane)
                plsc.store_scatter(eidx_c, [pos], pidx_v[IDXROWS + r, sl])
                plsc.store_scatter(ridx_c, [pos],
                                   lax.shift_right_logical(w0, 17))
                plsc.store_scatter(midx_c, [pos], loc)
                hidx = jnp.where(valid, loc, ACC_ROWS + lane)
                plsc.addupdate_scatter(
                    hist,
                    [lax.shift_right_logical(hidx, 4),
                     lax.bitwise_and(hidx, 15)],
                    ones16)
                return off + jnp.max(cs)

            off = compress
            for k in range(NSB * BLK // 16):
                sl = pl.ds(off + k * 16, 16)
                eidx_c[sl] = zero16
                ridx_c[sl] = zero16
                midx_c[sl] = dummy16

            gs = []
            for b in range(NSB):
                o = b * BLK
                gs.append(pltpu.async_copy(
                    ent_t.at[eidx_c.at[pl.ds(o, BLK)]], ent_w.at[b], sem_g))
                gs.append(pltpu.async_copy(
                    rel_t.at[ridx_c.at[pl.ds(o, BLK)]], rel_w.at[b], sem_g))
            for g_ in gs:
                g_.wait()
            ss = []
            for b in range(NSB):
                o = b * BLK
                ss.append(pltpu.async_copy(
                    ent_w.at[b], acc.at[midx_c.at[pl.ds(o, BLK)]],
                    sem_s, add=True))
                ss.append(pltpu.async_copy(
                    rel_w.at[b], acc.at[midx_c.at[pl.ds(o, BLK)]],
                    sem_s, add=True))
            for s_ in ss:
                s_.wait()

            nb = (off + (BLK - 1)) // BLK

            @pl.loop(NSB, nb)
            def spill(b):
                o = b * BLK
                g1 = pltpu.async_copy(
                    ent_t.at[eidx_c.at[pl.ds(o, BLK)]], ent_w.at[0], sem_g)
                g2 = pltpu.async_copy(
                    rel_t.at[ridx_c.at[pl.ds(o, BLK)]], rel_w.at[0], sem_g)
                g1.wait()
                g2.wait()
                s1 = pltpu.async_copy(
                    ent_w.at[0], acc.at[midx_c.at[pl.ds(o, BLK)]],
                    sem_s, add=True)
                s2 = pltpu.async_copy(
                    rel_w.at[0], acc.at[midx_c.at[pl.ds(o, BLK)]],
                    sem_s, add=True)
                s1.wait()
                s2.wait()

        for b in range(CNT_ROWS // 128):
            pltpu.async_copy(hist.at[pl.ds(b * 128, 128)],
                             cacc.at[iota_v.at[pl.ds(b * 128, 128)]],
                             sem_s, add=True).wait()

        plsc.subcore_barrier()
        pltpu.sync_copy(acc.at[pl.ds(s * ROWS_PER_SUB, ROWS_PER_SUB)],
                        msg_out.at[pl.ds(q * ACC_ROWS + s * ROWS_PER_SUB,
                                         ROWS_PER_SUB)])

        @pl.when(s == 0)
        def _():
            pltpu.sync_copy(cacc, cnt_out.at[pl.ds(q * CNT_ROWS, CNT_ROWS)])

        plsc.subcore_barrier()


def _sc_aggregate(ent_t, rel_t, pidx, zmsg, zcnt):
    mesh = plsc.VectorSubcoreMesh(core_axis_name="c", subcore_axis_name="s")
    fn = pl.kernel(
        _sc_body,
        out_type=(
            jax.ShapeDtypeStruct((4 * ACC_ROWS, D), jnp.float32),
            jax.ShapeDtypeStruct((4 * CNT_ROWS, 16), jnp.float32),
        ),
        mesh=mesh,
        compiler_params=pltpu.CompilerParams(use_tc_tiling_on_sc=False,
                                             needs_layout_passes=False),
        scratch_types=[
            pltpu.VMEM((2 * IDXROWS, 128), jnp.int32),
            pltpu.VMEM((CH + NSB * BLK + 32,), jnp.int32),
            pltpu.VMEM((CH + NSB * BLK + 32,), jnp.int32),
            pltpu.VMEM((CH + NSB * BLK + 32,), jnp.int32),
            pltpu.VMEM((NSB, BLK, D), jnp.float32),
            pltpu.VMEM((NSB, BLK, D), jnp.float32),
            pltpu.VMEM((CNT_ROWS, 16), jnp.float32),
            pltpu.VMEM((CNT_ROWS,), jnp.int32),
            pltpu.VMEM_SHARED((ACC_ROWS, D), jnp.float32),
            pltpu.VMEM_SHARED((CNT_ROWS, 16), jnp.float32),
            pltpu.SemaphoreType.DMA,
            pltpu.SemaphoreType.DMA,
        ],
    )
    return fn(ent_t, rel_t, pidx, zmsg, zcnt)


def _finish_body(item_ref, agg_ref, cnt_ref, intent_ref, out_ref):
    item = item_ref[...]
    intent = intent_ref[...]
    logits = jnp.dot(item, intent.T, preferred_element_type=jnp.float32)
    m = jnp.max(logits, axis=1, keepdims=True)
    e = jnp.exp(logits - m)
    att = e / jnp.sum(e, axis=1, keepdims=True)
    intent_item = jnp.dot(att, intent, preferred_element_type=jnp.float32)
    kg = agg_ref[...] / jnp.maximum(cnt_ref[...], 1.0)
    out_ref[...] = item + ALPHA * kg + BETA * intent_item


def _finish(item_emb, agg, cnt, intent_emb):
    n, d = item_emb.shape
    blk = 1000
    grid = (n // blk,)
    return pl.pallas_call(
        _finish_body,
        out_shape=jax.ShapeDtypeStruct((n, d), jnp.float32),
        grid=grid,
        in_specs=[
            pl.BlockSpec((blk, d), lambda i: (i, 0)),
            pl.BlockSpec((blk, d), lambda i: (i, 0)),
            pl.BlockSpec((blk, 1), lambda i: (i, 0)),
            pl.BlockSpec((4, d), lambda i: (0, 0)),
        ],
        out_specs=pl.BlockSpec((blk, d), lambda i: (i, 0)),
    )(item_emb, agg, cnt, intent_emb)


def kernel(edge_item_idx, edge_rel_idx, edge_ent_idx, user_emb, item_emb,
           entity_emb, relation_emb, intent_emb):
    f32 = jnp.float32
    i32 = jnp.int32
    pad = EPAD - E
    w0 = jnp.bitwise_or(edge_item_idx.astype(i32),
                        jnp.left_shift(edge_rel_idx.astype(i32), 17))
    w02 = jnp.concatenate(
        [w0, jnp.full((pad,), 0x1FFFF, i32)]).reshape(NCHUNKS, 1, CH)
    eidx2 = jnp.concatenate(
        [edge_ent_idx.astype(i32), jnp.zeros((pad,), i32)]
    ).reshape(NCHUNKS, 1, CH)
    pidx = jnp.concatenate([w02, eidx2], axis=1).reshape(
        2 * NCHUNKS * (CH // 128), 128)
    zmsg = jnp.zeros((ROWS_PER_SUB, D), f32)
    zcnt = jnp.zeros((CNT_ROWS, 16), f32)

    msg, cntp = _sc_aggregate(entity_emb, relation_emb, pidx, zmsg, zcnt)

    agg = jnp.concatenate(
        [msg[q * ACC_ROWS:q * ACC_ROWS + QUARTER] for q in range(4)], axis=0)
    cflat = cntp.reshape(4, CNT_ROWS * 16)
    cnt = jnp.concatenate([cflat[q, :QUARTER] for q in range(4)])[:, None]

    item_out = _finish(item_emb, agg, cnt, intent_emb)
    return (user_emb, item_out)


# speedup vs baseline: 2.1747x; 2.1747x over previous
import jax
import jax.numpy as jnp
from jax import lax
from jax.experimental import pallas as pl
from jax.experimental.pallas import tpu as pltpu
from jax.experimental.pallas import tpu_sc as plsc

E = 800_000
D = 64
CH = 1024
BLK = 128
NCHUNKS = 800
EPAD = NCHUNKS * CH
CHUNKS_PER_SUB = NCHUNKS // 16
QUARTER = 12_500
NPASS = 2
ACC_ROWS = 12_544
ROWS_PER_SUB = ACC_ROWS // 16
CNT_ROWS = 896
IDXROWS = CH // 128
NSB = 3
ALPHA = 0.6
BETA = 0.3


def _sc_body(ent_t, rel_t, pidx_h, zmsg_h, zcnt_h,
             msg_out, cnt_out,
             pidx_v, eidx_c, ridx_c, midx_c,
             ent_w, rel_tab, hist, iota_v, acc, cacc, sem_g, sem_s):
    c = lax.axis_index("c")
    s = lax.axis_index("s")
    chunk0 = s * CHUNKS_PER_SUB
    i32 = jnp.int32
    lane = lax.iota(i32, 16)
    ones16 = jnp.full((16,), 1.0, jnp.float32)
    zero16 = jnp.zeros((16,), i32)
    dummy16 = jnp.full((16,), QUARTER, i32)

    for k in range(CNT_ROWS // 16):
        iota_v[pl.ds(k * 16, 16)] = lane + (k * 16)

    pltpu.sync_copy(rel_t, rel_tab)

    for p in range(NPASS):
        q = c * NPASS + p
        base = q * QUARTER

        pltpu.sync_copy(zmsg_h, acc.at[pl.ds(s * ROWS_PER_SUB, ROWS_PER_SUB)])
        pltpu.sync_copy(zcnt_h, hist)

        @pl.when(s == 0)
        def _():
            pltpu.sync_copy(zcnt_h, cacc)

        plsc.subcore_barrier()

        @pl.loop(0, CHUNKS_PER_SUB)
        def chunkloop(g):
            chunk = chunk0 + g
            pltpu.sync_copy(pidx_h.at[pl.ds(2 * IDXROWS * chunk, 2 * IDXROWS)],
                            pidx_v)

            @pl.loop(0, CH // 16, init_carry=0)
            def compress(k, off):
                r = k // 8
                sl = pl.ds((k % 8) * 16, 16)
                w0 = pidx_v[r, sl]
                loc = lax.bitwise_and(w0, 0x1FFFF) - base
                valid = (loc >= 0) & (loc < QUARTER)
                cs = plsc.cumsum(jnp.where(valid, 1, 0))
                pos = jnp.where(valid, off + cs - 1, CH + NSB * BLK + lane)
                plsc.store_scatter(eidx_c, [pos], pidx_v[IDXROWS + r, sl])
                plsc.store_scatter(ridx_c, [pos],
                                   lax.shift_right_logical(w0, 17))
                plsc.store_scatter(midx_c, [pos], loc)
                hidx = jnp.where(valid, loc, ACC_ROWS + lane)
                plsc.addupdate_scatter(
                    hist,
                    [lax.shift_right_logical(hidx, 4),
                     lax.bitwise_and(hidx, 15)],
                    ones16)
                return off + jnp.max(cs)

            off = compress
            for k in range(NSB * BLK // 16):
                sl = pl.ds(off + k * 16, 16)
                eidx_c[sl] = zero16
                ridx_c[sl] = zero16
                midx_c[sl] = dummy16

            nb = jnp.maximum((off + (BLK - 1)) // BLK, 1)

            @pl.loop(0, nb)
            def blocks(b):
                o = b * BLK
                pltpu.async_copy(ent_t.at[eidx_c.at[pl.ds(o, BLK)]],
                                 ent_w.at[0], sem_g).wait()

                @pl.loop(0, BLK, unroll=2)
                def addrel(j):
                    ri = ridx_c[pl.ds(o + j, 16)][0]
                    for t in range(D // 16):
                        sl = pl.ds(t * 16, 16)
                        ent_w[0, j, sl] = ent_w[0, j, sl] + rel_tab[ri, sl]

                pltpu.async_copy(ent_w.at[0],
                                 acc.at[midx_c.at[pl.ds(o, BLK)]],
                                 sem_s, add=True).wait()

        for b in range(CNT_ROWS // 128):
            pltpu.async_copy(hist.at[pl.ds(b * 128, 128)],
                             cacc.at[iota_v.at[pl.ds(b * 128, 128)]],
                             sem_s, add=True).wait()

        plsc.subcore_barrier()
        pltpu.sync_copy(acc.at[pl.ds(s * ROWS_PER_SUB, ROWS_PER_SUB)],
                        msg_out.at[pl.ds(q * ACC_ROWS + s * ROWS_PER_SUB,
                                         ROWS_PER_SUB)])

        @pl.when(s == 0)
        def _():
            pltpu.sync_copy(cacc, cnt_out.at[pl.ds(q * CNT_ROWS, CNT_ROWS)])

        plsc.subcore_barrier()


def _sc_aggregate(ent_t, rel_t, pidx, zmsg, zcnt):
    mesh = plsc.VectorSubcoreMesh(core_axis_name="c", subcore_axis_name="s")
    fn = pl.kernel(
        _sc_body,
        out_type=(
            jax.ShapeDtypeStruct((4 * ACC_ROWS, D), jnp.float32),
            jax.ShapeDtypeStruct((4 * CNT_ROWS, 16), jnp.float32),
        ),
        mesh=mesh,
        compiler_params=pltpu.CompilerParams(use_tc_tiling_on_sc=False,
                                             needs_layout_passes=False),
        scratch_types=[
            pltpu.VMEM((2 * IDXROWS, 128), jnp.int32),
            pltpu.VMEM((CH + NSB * BLK + 32,), jnp.int32),
            pltpu.VMEM((CH + NSB * BLK + 32,), jnp.int32),
            pltpu.VMEM((CH + NSB * BLK + 32,), jnp.int32),
            pltpu.VMEM((1, BLK, D), jnp.float32),
            pltpu.VMEM((32, D), jnp.float32),
            pltpu.VMEM((CNT_ROWS, 16), jnp.float32),
            pltpu.VMEM((CNT_ROWS,), jnp.int32),
            pltpu.VMEM_SHARED((ACC_ROWS, D), jnp.float32),
            pltpu.VMEM_SHARED((CNT_ROWS, 16), jnp.float32),
            pltpu.SemaphoreType.DMA,
            pltpu.SemaphoreType.DMA,
        ],
    )
    return fn(ent_t, rel_t, pidx, zmsg, zcnt)


def _finish_body(item_ref, agg_ref, cnt_ref, intent_ref, out_ref):
    item = item_ref[...]
    intent = intent_ref[...]
    logits = jnp.dot(item, intent.T, preferred_element_type=jnp.float32)
    m = jnp.max(logits, axis=1, keepdims=True)
    e = jnp.exp(logits - m)
    att = e / jnp.sum(e, axis=1, keepdims=True)
    intent_item = jnp.dot(att, intent, preferred_element_type=jnp.float32)
    kg = agg_ref[...] / jnp.maximum(cnt_ref[...], 1.0)
    out_ref[...] = item + ALPHA * kg + BETA * intent_item


def _finish(item_emb, agg, cnt, intent_emb):
    n, d = item_emb.shape
    blk = 1000
    grid = (n // blk,)
    return pl.pallas_call(
        _finish_body,
        out_shape=jax.ShapeDtypeStruct((n, d), jnp.float32),
        grid=grid,
        in_specs=[
            pl.BlockSpec((blk, d), lambda i: (i, 0)),
            pl.BlockSpec((blk, d), lambda i: (i, 0)),
            pl.BlockSpec((blk, 1), lambda i: (i, 0)),
            pl.BlockSpec((4, d), lambda i: (0, 0)),
        ],
        out_specs=pl.BlockSpec((blk, d), lambda i: (i, 0)),
    )(item_emb, agg, cnt, intent_emb)


def kernel(edge_item_idx, edge_rel_idx, edge_ent_idx, user_emb, item_emb,
           entity_emb, relation_emb, intent_emb):
    f32 = jnp.float32
    i32 = jnp.int32
    pad = EPAD - E
    w0 = jnp.bitwise_or(edge_item_idx.astype(i32),
                        jnp.left_shift(edge_rel_idx.astype(i32), 17))
    w02 = jnp.concatenate(
        [w0, jnp.full((pad,), 0x1FFFF, i32)]).reshape(NCHUNKS, 1, CH)
    eidx2 = jnp.concatenate(
        [edge_ent_idx.astype(i32), jnp.zeros((pad,), i32)]
    ).reshape(NCHUNKS, 1, CH)
    pidx = jnp.concatenate([w02, eidx2], axis=1).reshape(
        2 * NCHUNKS * (CH // 128), 128)
    zmsg = jnp.zeros((ROWS_PER_SUB, D), f32)
    zcnt = jnp.zeros((CNT_ROWS, 16), f32)

    msg, cntp = _sc_aggregate(entity_emb, relation_emb, pidx, zmsg, zcnt)

    agg = jnp.concatenate(
        [msg[q * ACC_ROWS:q * ACC_ROWS + QUARTER] for q in range(4)], axis=0)
    cflat = cntp.reshape(4, CNT_ROWS * 16)
    cnt = jnp.concatenate([cflat[q, :QUARTER] for q in range(4)])[:, None]

    item_out = _finish(item_emb, agg, cnt, intent_emb)
    return (user_emb, item_out)
